# speedup vs baseline: 1.0806x; 1.0806x over previous
import functools

import jax
import jax.numpy as jnp
from jax import lax
from jax.experimental import pallas as pl
from jax.experimental.pallas import tpu as pltpu

_E = 16
_K = 2
_D = 1024
_DFF = 512
_S = 2048


def _moe_body(x_ref, gate_ref, wg_ref, wu_ref, wd_ref, o_ref, gat_ref, xbf_ref):
    e = pl.program_id(0)

    @pl.when(e == 0)
    def _router():
        x = x_ref[:]
        xbf_ref[:] = x.astype(jnp.bfloat16)
        scores = lax.dot_general(
            x, gate_ref[:], (((1,), (1,)), ((), ())),
            preferred_element_type=jnp.float32)
        iota = lax.broadcasted_iota(jnp.int32, (_S, _E), 1)
        m1 = jnp.max(scores, axis=-1, keepdims=True)
        i1 = jnp.min(jnp.where(scores == m1, iota, _E), axis=-1, keepdims=True)
        masked = jnp.where(iota == i1, -jnp.inf, scores)
        m2 = jnp.max(masked, axis=-1, keepdims=True)
        i2 = jnp.min(jnp.where(masked == m2, iota, _E), axis=-1, keepdims=True)
        t = jnp.exp(m2 - m1)
        p1 = 1.0 / (1.0 + t)
        p2 = 1.0 - p1
        gat_ref[:] = (jnp.where(iota == i1, p1, 0.0)
                      + jnp.where(iota == i2, p2, 0.0))

    xbf = xbf_ref[:]
    wg = wg_ref[0].astype(jnp.bfloat16)
    wu = wu_ref[0].astype(jnp.bfloat16)
    wd = wd_ref[0].astype(jnp.bfloat16)
    g = lax.dot_general(xbf, wg, (((1,), (1,)), ((), ())),
                        preferred_element_type=jnp.float32)
    u = lax.dot_general(xbf, wu, (((1,), (1,)), ((), ())),
                        preferred_element_type=jnp.float32)
    h = (g * (1.0 / (1.0 + jnp.exp(-g)))) * u
    out = lax.dot_general(h.astype(jnp.bfloat16), wd, (((1,), (1,)), ((), ())),
                          preferred_element_type=jnp.float32)
    onehot = (lax.broadcasted_iota(jnp.int32, (_E, 1), 0) == e).astype(jnp.float32)
    col = jnp.dot(gat_ref[:], onehot, preferred_element_type=jnp.float32)
    acc = col * out

    @pl.when(e == 0)
    def _init():
        o_ref[:] = acc

    @pl.when(e > 0)
    def _accum():
        o_ref[:] += acc


@functools.partial(jax.jit, static_argnames=("interpret",))
def _moe(x2d, gate_w, Wg, Wu, Wd, interpret=False):
    return pl.pallas_call(
        _moe_body,
        grid=(_E,),
        in_specs=[
            pl.BlockSpec((_S, _D), lambda e: (0, 0)),
            pl.BlockSpec((_E, _D), lambda e: (0, 0)),
            pl.BlockSpec((1, _DFF, _D), lambda e: (e, 0, 0)),
            pl.BlockSpec((1, _DFF, _D), lambda e: (e, 0, 0)),
            pl.BlockSpec((1, _D, _DFF), lambda e: (e, 0, 0)),
        ],
        out_specs=pl.BlockSpec((_S, _D), lambda e: (0, 0)),
        out_shape=jax.ShapeDtypeStruct((_S, _D), jnp.float32),
        scratch_shapes=[
            pltpu.VMEM((_S, _E), jnp.float32),
            pltpu.VMEM((_S, _D), jnp.bfloat16),
        ],
        interpret=interpret,
    )(x2d, gate_w, Wg, Wu, Wd)


def kernel(x, gate_w, Wg, Wu, Wd):
    b, s, d = x.shape
    y = _moe(x.reshape(s, d), gate_w, Wg, Wu, Wd)
    return y.reshape(b, s, d)
